# baseline (speedup 1.0000x reference)
import functools

import jax
import jax.numpy as jnp
from jax import lax
from jax.experimental import pallas as pl
from jax.experimental.pallas import tpu as pltpu
from jax.experimental.pallas import tpu_sc as plsc

_N = 10000
_D = 128
_NNZ = 320000
_B = 1024
_L = 50
_M = 10240
_LP = 64

_NC = 2
_NS = 16
_NW = _NC * _NS

_CH = 128
_SB = 4
_NSB = 20
_CPW = _SB * _NSB
_NCHP = _NW * _CPW
_NNZP = _NCHP * _CH

_PB = _B // _NW
_GT = _PB // 2

_VMESH = plsc.VectorSubcoreMesh(core_axis_name="c", subcore_axis_name="s")


def _scatter_body(src_hbm, rows_hbm, cols_hbm, vals_hbm, out_hbm,
                  acc, gbuf, rows_i, cols_i, vals_i, gsem, ssem):
    c = lax.axis_index("c")
    s = lax.axis_index("s")
    w = c * _NS + s
    base = w * _NSB

    def _z(r, _):
        for d in range(8):
            gbuf[0, r, pl.ds(d * 16, 16)] = jnp.zeros((16,), jnp.float32)
        return 0
    lax.fori_loop(0, _CH, _z, 0, unroll=4)
    rows_per_sub = _M // _NS
    for k in range(rows_per_sub // _CH):
        pltpu.sync_copy(gbuf.at[0],
                        acc.at[pl.ds(s * rows_per_sub + k * _CH, _CH)])
    plsc.subcore_barrier()

    def _load_idx(slot, sb):
        sbid = base + sb
        pltpu.sync_copy(rows_hbm.at[sbid], rows_i.at[slot])
        pltpu.sync_copy(cols_hbm.at[sbid], cols_i.at[slot])
        pltpu.sync_copy(vals_hbm.at[sbid], vals_i.at[slot])

    _load_idx(0, 0)
    pltpu.async_copy(src_hbm.at[cols_i.at[0, 0]], gbuf.at[0], gsem)

    def _chunk(i, _):
        sb = i // _SB
        k = i - sb * _SB
        par = sb % 2
        b = i % 2

        pltpu.make_async_copy(src_hbm.at[cols_i.at[par, k]],
                              gbuf.at[b], gsem).wait()

        @pl.when(i > 0)
        def _():
            pltpu.make_async_copy(src_hbm.at[pl.ds(0, _CH)],
                                  gbuf.at[1 - b], ssem).wait()

        @pl.when((k == 0) & (sb + 1 < _NSB))
        def _():
            _load_idx(1 - par, sb + 1)

        @pl.when(i + 1 < _CPW)
        def _():
            i1 = i + 1
            sb1 = i1 // _SB
            pltpu.async_copy(
                src_hbm.at[cols_i.at[sb1 % 2, i1 - sb1 * _SB]],
                gbuf.at[1 - b], gsem)

        def _scale(g, _):
            val16 = vals_i[par, k, pl.ds(g * 16, 16)]
            for l in range(16):
                v16 = jnp.full((16,), val16[l])
                e = g * 16 + l
                for d in range(8):
                    gbuf[b, e, pl.ds(d * 16, 16)] = (
                        gbuf[b, e, pl.ds(d * 16, 16)] * v16)
            return 0
        lax.fori_loop(0, _CH // 16, _scale, 0)

        pltpu.async_copy(gbuf.at[b], acc.at[rows_i.at[par, k]],
                         ssem, add=True)
        return 0
    lax.fori_loop(0, _CPW, _chunk, 0)
    pltpu.make_async_copy(src_hbm.at[pl.ds(0, _CH)],
                          gbuf.at[(_CPW - 1) % 2], ssem).wait()
    plsc.subcore_barrier()

    for k in range(rows_per_sub // _CH):
        r0 = s * rows_per_sub + k * _CH
        pltpu.sync_copy(acc.at[pl.ds(r0, _CH)], gbuf.at[0])
        pltpu.sync_copy(gbuf.at[0], out_hbm.at[c].at[pl.ds(r0, _CH)])


_scatter_kernel = functools.partial(
    pl.kernel,
    out_type=jax.ShapeDtypeStruct((_NC, _M, _D), jnp.float32),
    mesh=_VMESH,
    scratch_types=[
        pltpu.VMEM_SHARED((_M, _D), jnp.float32),
        pltpu.VMEM((2, _CH, _D), jnp.float32),
        pltpu.VMEM((2, _SB, _CH), jnp.int32),
        pltpu.VMEM((2, _SB, _CH), jnp.int32),
        pltpu.VMEM((2, _SB, _CH), jnp.float32),
        pltpu.SemaphoreType.DMA,
        pltpu.SemaphoreType.DMA,
    ],
)(_scatter_body)


def _combine_body(a_ref, b_ref, o_ref):
    o_ref[...] = jnp.maximum(a_ref[...] + b_ref[...], 0.0)


def _mean_body(e_ref, e1_ref, a_ref, b_ref, o_ref):
    e2 = jnp.maximum(a_ref[...] + b_ref[...], 0.0)
    o_ref[...] = (e_ref[...] + e1_ref[...] + e2) * jnp.float32(1.0 / 3.0)


_BR = 1280

_combine_kernel = pl.pallas_call(
    _combine_body,
    out_shape=jax.ShapeDtypeStruct((_M, _D), jnp.float32),
    grid=(_M // _BR,),
    in_specs=[pl.BlockSpec((_BR, _D), lambda i: (i, 0))] * 2,
    out_specs=pl.BlockSpec((_BR, _D), lambda i: (i, 0)),
)

_mean_kernel = pl.pallas_call(
    _mean_body,
    out_shape=jax.ShapeDtypeStruct((_M, _D), jnp.float32),
    grid=(_M // _BR,),
    in_specs=[pl.BlockSpec((_BR, _D), lambda i: (i, 0))] * 4,
    out_specs=pl.BlockSpec((_BR, _D), lambda i: (i, 0)),
)


def _prep_body(x_ref, x2_ref, inv_ref, xg_ref):
    valid = (x_ref[...] != 0).astype(jnp.float32)
    cnt = jnp.maximum(jnp.sum(valid, axis=1, keepdims=True), 1.0)
    inv_ref[...] = jnp.broadcast_to(1.0 / cnt, (_B, 16))
    x2 = x2_ref[...]
    xg_ref[...] = jnp.where(x2 > 0, x2 - 1, _N)


_prep_kernel = pl.pallas_call(
    _prep_body,
    out_shape=[
        jax.ShapeDtypeStruct((_B, 16), jnp.float32),
        jax.ShapeDtypeStruct((_B // 2, _CH), jnp.int32),
    ],
)


def _gather_mean_body(tab_hbm, xg_hbm, inv_hbm, out_hbm,
                      xb_v, inv_v, gbuf, obuf, gsem):
    c = lax.axis_index("c")
    s = lax.axis_index("s")
    w = s * _NC + c
    pltpu.sync_copy(xg_hbm.at[pl.ds(w * _GT, _GT)], xb_v)
    pltpu.sync_copy(inv_hbm.at[pl.ds(w * _PB, _PB)], inv_v)

    zeros = jnp.zeros((16,), jnp.float32)

    for b in range(4):
        pltpu.async_copy(tab_hbm.at[xb_v.at[b]], gbuf.at[b], gsem)

    def _round(q, _):
        for b in range(4):
            j = 4 * q + b
            pltpu.make_async_copy(tab_hbm.at[xb_v.at[j]],
                                  gbuf.at[b], gsem).wait()
            for half in range(2):
                pat = 2 * j + half

                def _acc(e, accs):
                    return tuple(
                        a + gbuf[b, half * _LP + e, pl.ds(d * 16, 16)]
                        for d, a in enumerate(accs))
                accs = lax.fori_loop(0, _LP, _acc,
                                     tuple(zeros for _ in range(8)),
                                     unroll=4)
                inv = inv_v[pat]
                for d in range(8):
                    obuf[pat, pl.ds(d * 16, 16)] = accs[d] * inv

            @pl.when(j + 4 < _GT)
            def _():
                pltpu.async_copy(tab_hbm.at[xb_v.at[j + 4]],
                                 gbuf.at[b], gsem)
        return 0
    lax.fori_loop(0, _GT // 4, _round, 0)
    pltpu.sync_copy(obuf, out_hbm.at[pl.ds(w * _PB, _PB)])


_gather_mean_kernel = functools.partial(
    pl.kernel,
    out_type=jax.ShapeDtypeStruct((_B, _D), jnp.float32),
    mesh=_VMESH,
    scratch_types=[
        pltpu.VMEM((_GT, _CH), jnp.int32),
        pltpu.VMEM((_PB, 16), jnp.float32),
        pltpu.VMEM((4, _CH, _D), jnp.float32),
        pltpu.VMEM((_PB, _D), jnp.float32),
        pltpu.SemaphoreType.DMA,
    ],
)(_gather_mean_body)


@jax.jit
def kernel(x, adj_indices, adj_values, embedding):
    npad = _NNZP - _NNZ
    pad_ids = (jnp.arange(npad, dtype=jnp.int32) * 97) % _N
    sb_shape = (_NCHP // _SB, _SB, _CH)
    rows2 = jnp.concatenate([adj_indices[0], pad_ids]).reshape(sb_shape)
    cols2 = jnp.concatenate([adj_indices[1], pad_ids]).reshape(sb_shape)
    vals2 = jnp.concatenate(
        [adj_values, jnp.zeros((npad,), jnp.float32)]).reshape(sb_shape)

    e0 = jnp.zeros((_M, _D), jnp.float32).at[:_N].set(embedding[1:])
    x_pad = jnp.zeros((_B, _LP), jnp.int32).at[:, :_L].set(x)
    x_pad2 = x_pad.reshape(_B // 2, _CH)

    p1 = _scatter_kernel(e0, rows2, cols2, vals2)
    e1 = _combine_kernel(p1[0], p1[1])
    p2 = _scatter_kernel(e1, rows2, cols2, vals2)
    eavg = _mean_kernel(e0, e1, p2[0], p2[1])
    invb, xg = _prep_kernel(x_pad, x_pad2)
    return _gather_mean_kernel(eavg, xg, invb)


# speedup vs baseline: 1.6514x; 1.6514x over previous
import functools

import jax
import jax.numpy as jnp
from jax import lax
from jax.experimental import pallas as pl
from jax.experimental.pallas import tpu as pltpu
from jax.experimental.pallas import tpu_sc as plsc

_N = 10000
_D = 128
_NNZ = 320000
_B = 1024
_L = 50
_M = 10240
_LP = 64

_NC = 2
_NS = 16
_NW = _NC * _NS

_CH = 128
_SB = 4
_NSB = 20
_CPW = _SB * _NSB
_NCHP = _NW * _CPW
_NNZP = _NCHP * _CH

_PB = _B // _NW
_GT = _PB // 2

_VMESH = plsc.VectorSubcoreMesh(core_axis_name="c", subcore_axis_name="s")


def _scatter_body(src_hbm, rows_hbm, cols_hbm, vals_hbm, out_hbm,
                  acc, gbuf, rows_i, cols_i, vals_i, gsem, ssem):
    c = lax.axis_index("c")
    s = lax.axis_index("s")
    w = c * _NS + s
    base = w * _NSB

    def _z(r, _):
        for d in range(8):
            gbuf[0, r, pl.ds(d * 16, 16)] = jnp.zeros((16,), jnp.float32)
        return 0
    lax.fori_loop(0, _CH, _z, 0, unroll=4)
    rows_per_sub = _M // _NS
    for k in range(rows_per_sub // _CH):
        pltpu.sync_copy(gbuf.at[0],
                        acc.at[pl.ds(s * rows_per_sub + k * _CH, _CH)])
    plsc.subcore_barrier()

    def _load_idx(slot, sb):
        sbid = base + sb
        pltpu.sync_copy(rows_hbm.at[sbid], rows_i.at[slot])
        pltpu.sync_copy(cols_hbm.at[sbid], cols_i.at[slot])
        pltpu.sync_copy(vals_hbm.at[sbid], vals_i.at[slot])

    _load_idx(0, 0)
    pltpu.async_copy(src_hbm.at[cols_i.at[0, 0]], gbuf.at[0], gsem)

    def _do_sb(par, sb):
        for k in range(_SB):
            b = k % 2
            i = sb * _SB + k
            pltpu.make_async_copy(src_hbm.at[cols_i.at[par, k]],
                                  gbuf.at[b], gsem).wait()

            @pl.when(i > 0)
            def _():
                pltpu.make_async_copy(src_hbm.at[pl.ds(0, _CH)],
                                      gbuf.at[1 - b], ssem).wait()

            if k == 0:
                @pl.when(sb + 1 < _NSB)
                def _():
                    _load_idx(1 - par, sb + 1)

            if k < _SB - 1:
                pltpu.async_copy(src_hbm.at[cols_i.at[par, k + 1]],
                                 gbuf.at[1 - b], gsem)
            else:
                @pl.when(sb + 1 < _NSB)
                def _():
                    pltpu.async_copy(src_hbm.at[cols_i.at[1 - par, 0]],
                                     gbuf.at[0], gsem)

            def _scale(g, _):
                val16 = vals_i[par, k, pl.ds(g * 16, 16)]
                for l in range(16):
                    v16 = jnp.full((16,), val16[l])
                    e = g * 16 + l
                    for d in range(8):
                        gbuf[b, e, pl.ds(d * 16, 16)] = (
                            gbuf[b, e, pl.ds(d * 16, 16)] * v16)
                return 0
            lax.fori_loop(0, _CH // 16, _scale, 0)

            pltpu.async_copy(gbuf.at[b], acc.at[rows_i.at[par, k]],
                             ssem, add=True)

    def _pair(q, _):
        _do_sb(0, 2 * q)
        _do_sb(1, 2 * q + 1)
        return 0
    lax.fori_loop(0, _NSB // 2, _pair, 0)
    pltpu.make_async_copy(src_hbm.at[pl.ds(0, _CH)],
                          gbuf.at[(_CPW - 1) % 2], ssem).wait()
    plsc.subcore_barrier()

    for k in range(rows_per_sub // _CH):
        r0 = s * rows_per_sub + k * _CH
        pltpu.sync_copy(acc.at[pl.ds(r0, _CH)], gbuf.at[0])
        pltpu.sync_copy(gbuf.at[0], out_hbm.at[c].at[pl.ds(r0, _CH)])


_scatter_kernel = functools.partial(
    pl.kernel,
    out_type=jax.ShapeDtypeStruct((_NC, _M, _D), jnp.float32),
    mesh=_VMESH,
    scratch_types=[
        pltpu.VMEM_SHARED((_M, _D), jnp.float32),
        pltpu.VMEM((2, _CH, _D), jnp.float32),
        pltpu.VMEM((2, _SB, _CH), jnp.int32),
        pltpu.VMEM((2, _SB, _CH), jnp.int32),
        pltpu.VMEM((2, _SB, _CH), jnp.float32),
        pltpu.SemaphoreType.DMA,
        pltpu.SemaphoreType.DMA,
    ],
)(_scatter_body)


def _combine_body(a_ref, b_ref, o_ref):
    o_ref[...] = jnp.maximum(a_ref[...] + b_ref[...], 0.0)


def _mean_body(e_ref, e1_ref, a_ref, b_ref, o_ref):
    e2 = jnp.maximum(a_ref[...] + b_ref[...], 0.0)
    o_ref[...] = (e_ref[...] + e1_ref[...] + e2) * jnp.float32(1.0 / 3.0)


_BR = 1280

_combine_kernel = pl.pallas_call(
    _combine_body,
    out_shape=jax.ShapeDtypeStruct((_M, _D), jnp.float32),
    grid=(_M // _BR,),
    in_specs=[pl.BlockSpec((_BR, _D), lambda i: (i, 0))] * 2,
    out_specs=pl.BlockSpec((_BR, _D), lambda i: (i, 0)),
)

_mean_kernel = pl.pallas_call(
    _mean_body,
    out_shape=jax.ShapeDtypeStruct((_M, _D), jnp.float32),
    grid=(_M // _BR,),
    in_specs=[pl.BlockSpec((_BR, _D), lambda i: (i, 0))] * 4,
    out_specs=pl.BlockSpec((_BR, _D), lambda i: (i, 0)),
)


def _prep_body(x_ref, x2_ref, inv_ref, xg_ref):
    valid = (x_ref[...] != 0).astype(jnp.float32)
    cnt = jnp.maximum(jnp.sum(valid, axis=1, keepdims=True), 1.0)
    inv_ref[...] = jnp.broadcast_to(1.0 / cnt, (_B, 16))
    x2 = x2_ref[...]
    xg_ref[...] = jnp.where(x2 > 0, x2 - 1, _N)


_prep_kernel = pl.pallas_call(
    _prep_body,
    out_shape=[
        jax.ShapeDtypeStruct((_B, 16), jnp.float32),
        jax.ShapeDtypeStruct((_B // 2, _CH), jnp.int32),
    ],
)


def _gather_mean_body(tab_hbm, xg_hbm, inv_hbm, out_hbm,
                      xb_v, inv_v, gbuf, obuf, gsem):
    c = lax.axis_index("c")
    s = lax.axis_index("s")
    w = s * _NC + c
    pltpu.sync_copy(xg_hbm.at[pl.ds(w * _GT, _GT)], xb_v)
    pltpu.sync_copy(inv_hbm.at[pl.ds(w * _PB, _PB)], inv_v)

    zeros = jnp.zeros((16,), jnp.float32)

    for b in range(4):
        pltpu.async_copy(tab_hbm.at[xb_v.at[b]], gbuf.at[b], gsem)

    for j in range(_GT):
        b = j % 4
        pltpu.make_async_copy(tab_hbm.at[xb_v.at[j]],
                              gbuf.at[b], gsem).wait()
        for half in range(2):
            pat = 2 * j + half

            def _acc(e, accs):
                return tuple(
                    a + gbuf[b, half * _LP + e, pl.ds(d * 16, 16)]
                    for d, a in enumerate(accs))
            accs = lax.fori_loop(0, _LP, _acc,
                                 tuple(zeros for _ in range(8)),
                                 unroll=4)
            inv = inv_v[pat]
            for d in range(8):
                obuf[pat, pl.ds(d * 16, 16)] = accs[d] * inv

        if j + 4 < _GT:
            pltpu.async_copy(tab_hbm.at[xb_v.at[j + 4]],
                             gbuf.at[b], gsem)
    pltpu.sync_copy(obuf, out_hbm.at[pl.ds(w * _PB, _PB)])


_gather_mean_kernel = functools.partial(
    pl.kernel,
    out_type=jax.ShapeDtypeStruct((_B, _D), jnp.float32),
    mesh=_VMESH,
    scratch_types=[
        pltpu.VMEM((_GT, _CH), jnp.int32),
        pltpu.VMEM((_PB, 16), jnp.float32),
        pltpu.VMEM((4, _CH, _D), jnp.float32),
        pltpu.VMEM((_PB, _D), jnp.float32),
        pltpu.SemaphoreType.DMA,
    ],
)(_gather_mean_body)


@jax.jit
def kernel(x, adj_indices, adj_values, embedding):
    npad = _NNZP - _NNZ
    pad_ids = (jnp.arange(npad, dtype=jnp.int32) * 97) % _N
    sb_shape = (_NCHP // _SB, _SB, _CH)
    rows2 = jnp.concatenate([adj_indices[0], pad_ids]).reshape(sb_shape)
    cols2 = jnp.concatenate([adj_indices[1], pad_ids]).reshape(sb_shape)
    vals2 = jnp.concatenate(
        [adj_values, jnp.zeros((npad,), jnp.float32)]).reshape(sb_shape)

    e0 = jnp.zeros((_M, _D), jnp.float32).at[:_N].set(embedding[1:])
    x_pad = jnp.zeros((_B, _LP), jnp.int32).at[:, :_L].set(x)
    x_pad2 = x_pad.reshape(_B // 2, _CH)

    p1 = _scatter_kernel(e0, rows2, cols2, vals2)
    e1 = _combine_kernel(p1[0], p1[1])
    p2 = _scatter_kernel(e1, rows2, cols2, vals2)
    eavg = _mean_kernel(e0, e1, p2[0], p2[1])
    invb, xg = _prep_kernel(x_pad, x_pad2)
    return _gather_mean_kernel(eavg, xg, invb)


# speedup vs baseline: 3.5353x; 2.1408x over previous
import functools

import jax
import jax.numpy as jnp
from jax import lax
from jax.experimental import pallas as pl
from jax.experimental.pallas import tpu as pltpu
from jax.experimental.pallas import tpu_sc as plsc

_N = 10000
_D = 128
_NNZ = 320000
_B = 1024
_L = 50
_M = 10240
_LP = 64

_NC = 2
_NS = 16
_NW = _NC * _NS

_CH = 128
_SB = 4
_NSB = 20
_CPW = _SB * _NSB
_NCHP = _NW * _CPW
_NNZP = _NCHP * _CH

_PB = _B // _NW
_GT = _PB // 2

_VMESH = plsc.VectorSubcoreMesh(core_axis_name="c", subcore_axis_name="s")


def _scatter_body(src_hbm, rows_hbm, cols_hbm, vals_hbm, out_hbm,
                  acc, gbuf, rows_i, cols_i, vals_i, gsem, ssem):
    c = lax.axis_index("c")
    s = lax.axis_index("s")
    w = c * _NS + s
    base = w * _NSB

    def _z(r, _):
        for d in range(8):
            gbuf[0, r, pl.ds(d * 16, 16)] = jnp.zeros((16,), jnp.float32)
        return 0
    lax.fori_loop(0, _CH, _z, 0, unroll=4)
    rows_per_sub = _M // _NS
    for k in range(rows_per_sub // _CH):
        pltpu.sync_copy(gbuf.at[0],
                        acc.at[pl.ds(s * rows_per_sub + k * _CH, _CH)])
    plsc.subcore_barrier()

    def _load_idx(slot, sb):
        sbid = base + sb
        pltpu.sync_copy(rows_hbm.at[sbid], rows_i.at[slot])
        pltpu.sync_copy(cols_hbm.at[sbid], cols_i.at[slot])
        pltpu.sync_copy(vals_hbm.at[sbid], vals_i.at[slot])

    _load_idx(0, 0)
    pltpu.async_copy(src_hbm.at[cols_i.at[0, 0]], gbuf.at[0], gsem)

    def _do_sb(par, sb):
        for k in range(_SB):
            b = k % 2
            i = sb * _SB + k
            pltpu.make_async_copy(src_hbm.at[cols_i.at[par, k]],
                                  gbuf.at[b], gsem).wait()

            @pl.when(i > 0)
            def _():
                pltpu.make_async_copy(src_hbm.at[pl.ds(0, _CH)],
                                      gbuf.at[1 - b], ssem).wait()

            if k == 0:
                @pl.when(sb + 1 < _NSB)
                def _():
                    _load_idx(1 - par, sb + 1)

            if k < _SB - 1:
                pltpu.async_copy(src_hbm.at[cols_i.at[par, k + 1]],
                                 gbuf.at[1 - b], gsem)
            else:
                @pl.when(sb + 1 < _NSB)
                def _():
                    pltpu.async_copy(src_hbm.at[cols_i.at[1 - par, 0]],
                                     gbuf.at[0], gsem)

            def _scale(g, _):
                val16 = vals_i[par, k, pl.ds(g * 16, 16)]
                for l in range(16):
                    v16 = jnp.full((16,), val16[l])
                    e = g * 16 + l
                    for d in range(8):
                        gbuf[b, e, pl.ds(d * 16, 16)] = (
                            gbuf[b, e, pl.ds(d * 16, 16)] * v16)
                return 0
            lax.fori_loop(0, _CH // 16, _scale, 0)

            pltpu.async_copy(gbuf.at[b], acc.at[rows_i.at[par, k]],
                             ssem, add=True)

    def _pair(q, _):
        _do_sb(0, 2 * q)
        _do_sb(1, 2 * q + 1)
        return 0
    lax.fori_loop(0, _NSB // 2, _pair, 0)
    pltpu.make_async_copy(src_hbm.at[pl.ds(0, _CH)],
                          gbuf.at[(_CPW - 1) % 2], ssem).wait()
    plsc.subcore_barrier()

    for k in range(rows_per_sub // _CH):
        r0 = s * rows_per_sub + k * _CH
        pltpu.sync_copy(acc.at[pl.ds(r0, _CH)], gbuf.at[0])
        pltpu.sync_copy(gbuf.at[0], out_hbm.at[c].at[pl.ds(r0, _CH)])


_scatter_kernel = functools.partial(
    pl.kernel,
    out_type=jax.ShapeDtypeStruct((_NC, _M, _D), jnp.float32),
    mesh=_VMESH,
    scratch_types=[
        pltpu.VMEM_SHARED((_M, _D), jnp.float32),
        pltpu.VMEM((2, _CH, _D), jnp.float32),
        pltpu.VMEM((2, _SB, _CH), jnp.int32),
        pltpu.VMEM((2, _SB, _CH), jnp.int32),
        pltpu.VMEM((2, _SB, _CH), jnp.float32),
        pltpu.SemaphoreType.DMA,
        pltpu.SemaphoreType.DMA,
    ],
)(_scatter_body)


def _combine_body(a_ref, b_ref, o_ref):
    o_ref[...] = jnp.maximum(a_ref[...] + b_ref[...], 0.0)


def _mean_body(e_ref, e1_ref, a_ref, b_ref, o_ref):
    e2 = jnp.maximum(a_ref[...] + b_ref[...], 0.0)
    o_ref[...] = (e_ref[...] + e1_ref[...] + e2) * jnp.float32(1.0 / 3.0)


_BR = 1280

_combine_kernel = pl.pallas_call(
    _combine_body,
    out_shape=jax.ShapeDtypeStruct((_M, _D), jnp.float32),
    grid=(_M // _BR,),
    in_specs=[pl.BlockSpec((_BR, _D), lambda i: (i, 0))] * 2,
    out_specs=pl.BlockSpec((_BR, _D), lambda i: (i, 0)),
)

_mean_kernel = pl.pallas_call(
    _mean_body,
    out_shape=jax.ShapeDtypeStruct((_M, _D), jnp.float32),
    grid=(_M // _BR,),
    in_specs=[pl.BlockSpec((_BR, _D), lambda i: (i, 0))] * 4,
    out_specs=pl.BlockSpec((_BR, _D), lambda i: (i, 0)),
)


def _prep_body(x_ref, x2_ref, inv_ref, xg_ref):
    valid = (x_ref[...] != 0).astype(jnp.float32)
    cnt = jnp.maximum(jnp.sum(valid, axis=1, keepdims=True), 1.0)
    inv_ref[...] = jnp.broadcast_to(1.0 / cnt, (_B, 16))
    x2 = x2_ref[...]
    spread = _N + jax.lax.broadcasted_iota(jnp.int32, x2.shape, 1) % (_M - _N)
    xg_ref[...] = jnp.where(x2 > 0, x2 - 1, spread)


_prep_kernel = pl.pallas_call(
    _prep_body,
    out_shape=[
        jax.ShapeDtypeStruct((_B, 16), jnp.float32),
        jax.ShapeDtypeStruct((_B // 2, _CH), jnp.int32),
    ],
)


def _gather_mean_body(tab_hbm, xg_hbm, inv_hbm, out_hbm,
                      xb_v, inv_v, gbuf, obuf, gsem):
    c = lax.axis_index("c")
    s = lax.axis_index("s")
    w = s * _NC + c
    pltpu.sync_copy(xg_hbm.at[pl.ds(w * _GT, _GT)], xb_v)
    pltpu.sync_copy(inv_hbm.at[pl.ds(w * _PB, _PB)], inv_v)

    zeros = jnp.zeros((16,), jnp.float32)

    for b in range(4):
        pltpu.async_copy(tab_hbm.at[xb_v.at[b]], gbuf.at[b], gsem)

    for j in range(_GT):
        b = j % 4
        pltpu.make_async_copy(tab_hbm.at[xb_v.at[j]],
                              gbuf.at[b], gsem).wait()
        for half in range(2):
            pat = 2 * j + half

            def _acc(e, accs):
                return tuple(
                    a + gbuf[b, half * _LP + e, pl.ds(d * 16, 16)]
                    for d, a in enumerate(accs))
            accs = lax.fori_loop(0, _LP, _acc,
                                 tuple(zeros for _ in range(8)),
                                 unroll=4)
            inv = inv_v[pat]
            for d in range(8):
                obuf[pat, pl.ds(d * 16, 16)] = accs[d] * inv

        if j + 4 < _GT:
            pltpu.async_copy(tab_hbm.at[xb_v.at[j + 4]],
                             gbuf.at[b], gsem)
    pltpu.sync_copy(obuf, out_hbm.at[pl.ds(w * _PB, _PB)])


_gather_mean_kernel = functools.partial(
    pl.kernel,
    out_type=jax.ShapeDtypeStruct((_B, _D), jnp.float32),
    mesh=_VMESH,
    scratch_types=[
        pltpu.VMEM((_GT, _CH), jnp.int32),
        pltpu.VMEM((_PB, 16), jnp.float32),
        pltpu.VMEM((4, _CH, _D), jnp.float32),
        pltpu.VMEM((_PB, _D), jnp.float32),
        pltpu.SemaphoreType.DMA,
    ],
)(_gather_mean_body)


@jax.jit
def kernel(x, adj_indices, adj_values, embedding):
    npad = _NNZP - _NNZ
    pad_ids = (jnp.arange(npad, dtype=jnp.int32) * 97) % _N
    sb_shape = (_NCHP // _SB, _SB, _CH)
    rows2 = jnp.concatenate([adj_indices[0], pad_ids]).reshape(sb_shape)
    cols2 = jnp.concatenate([adj_indices[1], pad_ids]).reshape(sb_shape)
    vals2 = jnp.concatenate(
        [adj_values, jnp.zeros((npad,), jnp.float32)]).reshape(sb_shape)

    e0 = jnp.zeros((_M, _D), jnp.float32).at[:_N].set(embedding[1:])
    x_pad = jnp.zeros((_B, _LP), jnp.int32).at[:, :_L].set(x)
    x_pad2 = x_pad.reshape(_B // 2, _CH)

    p1 = _scatter_kernel(e0, rows2, cols2, vals2)
    e1 = _combine_kernel(p1[0], p1[1])
    p2 = _scatter_kernel(e1, rows2, cols2, vals2)
    eavg = _mean_kernel(e0, e1, p2[0], p2[1])
    invb, xg = _prep_kernel(x_pad, x_pad2)
    return _gather_mean_kernel(eavg, xg, invb)


# speedup vs baseline: 3.8954x; 1.1019x over previous
import functools

import jax
import jax.numpy as jnp
from jax import lax
from jax.experimental import pallas as pl
from jax.experimental.pallas import tpu as pltpu
from jax.experimental.pallas import tpu_sc as plsc

_N = 10000
_D = 128
_NNZ = 320000
_B = 1024
_L = 50
_M = 10240
_LP = 64

_NC = 2
_NS = 16
_NW = _NC * _NS

_CH = 128
_SB = 8
_NSB = 10
_CPW = _SB * _NSB
_NCHP = _NW * _CPW
_NNZP = _NCHP * _CH

_PB = _B // _NW
_GT = _PB // 2

_VMESH = plsc.VectorSubcoreMesh(core_axis_name="c", subcore_axis_name="s")


def _scatter_body(src_hbm, rows_hbm, cols_hbm, vals_hbm, out_hbm,
                  acc, gbuf, rows_i, cols_i, vals_i, gsem, ssem, isem):
    c = lax.axis_index("c")
    s = lax.axis_index("s")
    w = c * _NS + s
    base = w * _NSB

    def _z(r, _):
        for d in range(8):
            gbuf[0, r, pl.ds(d * 16, 16)] = jnp.zeros((16,), jnp.float32)
        return 0
    lax.fori_loop(0, _CH, _z, 0, unroll=4)
    rows_per_sub = _M // _NS
    for k in range(rows_per_sub // _CH):
        pltpu.sync_copy(gbuf.at[0],
                        acc.at[pl.ds(s * rows_per_sub + k * _CH, _CH)])
    plsc.subcore_barrier()

    def _fire_idx(slot, sb):
        sbid = base + sb
        pltpu.async_copy(rows_hbm.at[sbid], rows_i.at[slot], isem)
        pltpu.async_copy(cols_hbm.at[sbid], cols_i.at[slot], isem)
        pltpu.async_copy(vals_hbm.at[sbid], vals_i.at[slot], isem)

    def _drain_idx(slot, sb):
        sbid = base + sb
        pltpu.make_async_copy(rows_hbm.at[sbid], rows_i.at[slot],
                              isem).wait()
        pltpu.make_async_copy(cols_hbm.at[sbid], cols_i.at[slot],
                              isem).wait()
        pltpu.make_async_copy(vals_hbm.at[sbid], vals_i.at[slot],
                              isem).wait()

    _fire_idx(0, 0)
    _drain_idx(0, 0)
    pltpu.async_copy(src_hbm.at[cols_i.at[0, 0]], gbuf.at[0], gsem)

    def _do_sb(par, sb):
        for k in range(_SB):
            b = k % 2
            i = sb * _SB + k
            pltpu.make_async_copy(src_hbm.at[cols_i.at[par, k]],
                                  gbuf.at[b], gsem).wait()

            @pl.when(i > 0)
            def _():
                pltpu.make_async_copy(src_hbm.at[pl.ds(0, _CH)],
                                      gbuf.at[1 - b], ssem).wait()

            if k == 0:
                @pl.when(sb + 1 < _NSB)
                def _():
                    _fire_idx(1 - par, sb + 1)

            if k == _SB - 2:
                @pl.when(sb + 1 < _NSB)
                def _():
                    _drain_idx(1 - par, sb + 1)

            if k < _SB - 1:
                pltpu.async_copy(src_hbm.at[cols_i.at[par, k + 1]],
                                 gbuf.at[1 - b], gsem)
            else:
                @pl.when(sb + 1 < _NSB)
                def _():
                    pltpu.async_copy(src_hbm.at[cols_i.at[1 - par, 0]],
                                     gbuf.at[0], gsem)

            @plsc.parallel_loop(0, _CH // 16, unroll=2)
            def _scale(g):
                val16 = vals_i[par, k, pl.ds(g * 16, 16)]
                for l in range(16):
                    v16 = jnp.full((16,), val16[l])
                    e = g * 16 + l
                    for d in range(8):
                        gbuf[b, e, pl.ds(d * 16, 16)] = (
                            gbuf[b, e, pl.ds(d * 16, 16)] * v16)

            pltpu.async_copy(gbuf.at[b], acc.at[rows_i.at[par, k]],
                             ssem, add=True)

    def _pair(q, _):
        _do_sb(0, 2 * q)
        _do_sb(1, 2 * q + 1)
        return 0
    lax.fori_loop(0, _NSB // 2, _pair, 0)
    pltpu.make_async_copy(src_hbm.at[pl.ds(0, _CH)],
                          gbuf.at[(_CPW - 1) % 2], ssem).wait()
    plsc.subcore_barrier()

    for k in range(rows_per_sub // _CH):
        r0 = s * rows_per_sub + k * _CH
        pltpu.sync_copy(acc.at[pl.ds(r0, _CH)], gbuf.at[0])
        pltpu.sync_copy(gbuf.at[0], out_hbm.at[c].at[pl.ds(r0, _CH)])


_scatter_kernel = functools.partial(
    pl.kernel,
    out_type=jax.ShapeDtypeStruct((_NC, _M, _D), jnp.float32),
    mesh=_VMESH,
    scratch_types=[
        pltpu.VMEM_SHARED((_M, _D), jnp.float32),
        pltpu.VMEM((2, _CH, _D), jnp.float32),
        pltpu.VMEM((2, _SB, _CH), jnp.int32),
        pltpu.VMEM((2, _SB, _CH), jnp.int32),
        pltpu.VMEM((2, _SB, _CH), jnp.float32),
        pltpu.SemaphoreType.DMA,
        pltpu.SemaphoreType.DMA,
        pltpu.SemaphoreType.DMA,
    ],
)(_scatter_body)


def _combine_body(a_ref, b_ref, o_ref):
    o_ref[...] = jnp.maximum(a_ref[...] + b_ref[...], 0.0)


def _mean_body(e_ref, e1_ref, a_ref, b_ref, o_ref):
    e2 = jnp.maximum(a_ref[...] + b_ref[...], 0.0)
    o_ref[...] = (e_ref[...] + e1_ref[...] + e2) * jnp.float32(1.0 / 3.0)


_BR = 1280

_combine_kernel = pl.pallas_call(
    _combine_body,
    out_shape=jax.ShapeDtypeStruct((_M, _D), jnp.float32),
    grid=(_M // _BR,),
    in_specs=[pl.BlockSpec((_BR, _D), lambda i: (i, 0))] * 2,
    out_specs=pl.BlockSpec((_BR, _D), lambda i: (i, 0)),
)

_mean_kernel = pl.pallas_call(
    _mean_body,
    out_shape=jax.ShapeDtypeStruct((_M, _D), jnp.float32),
    grid=(_M // _BR,),
    in_specs=[pl.BlockSpec((_BR, _D), lambda i: (i, 0))] * 4,
    out_specs=pl.BlockSpec((_BR, _D), lambda i: (i, 0)),
)


def _prep_body(x_ref, x2_ref, inv_ref, xg_ref):
    valid = (x_ref[...] != 0).astype(jnp.float32)
    cnt = jnp.maximum(jnp.sum(valid, axis=1, keepdims=True), 1.0)
    inv_ref[...] = jnp.broadcast_to(1.0 / cnt, (_B, 16))
    x2 = x2_ref[...]
    spread = _N + jax.lax.broadcasted_iota(jnp.int32, x2.shape, 1) % (_M - _N)
    xg_ref[...] = jnp.where(x2 > 0, x2 - 1, spread)


_prep_kernel = pl.pallas_call(
    _prep_body,
    out_shape=[
        jax.ShapeDtypeStruct((_B, 16), jnp.float32),
        jax.ShapeDtypeStruct((_B // 2, _CH), jnp.int32),
    ],
)


def _gather_mean_body(tab_hbm, xg_hbm, inv_hbm, out_hbm,
                      xb_v, inv_v, gbuf, obuf, gsem):
    c = lax.axis_index("c")
    s = lax.axis_index("s")
    w = s * _NC + c
    pltpu.sync_copy(xg_hbm.at[pl.ds(w * _GT, _GT)], xb_v)
    pltpu.sync_copy(inv_hbm.at[pl.ds(w * _PB, _PB)], inv_v)

    zeros = jnp.zeros((16,), jnp.float32)

    for b in range(6):
        pltpu.async_copy(tab_hbm.at[xb_v.at[b]], gbuf.at[b], gsem)

    for j in range(_GT):
        b = j % 6
        pltpu.make_async_copy(tab_hbm.at[xb_v.at[j]],
                              gbuf.at[b], gsem).wait()
        for half in range(2):
            pat = 2 * j + half

            def _acc(e, accs):
                return tuple(
                    a + gbuf[b, half * _LP + e, pl.ds(d * 16, 16)]
                    for d, a in enumerate(accs))
            accs = lax.fori_loop(0, _LP, _acc,
                                 tuple(zeros for _ in range(8)),
                                 unroll=4)
            inv = inv_v[pat]
            for d in range(8):
                obuf[pat, pl.ds(d * 16, 16)] = accs[d] * inv

        if j + 6 < _GT:
            pltpu.async_copy(tab_hbm.at[xb_v.at[j + 6]],
                             gbuf.at[b], gsem)
    pltpu.sync_copy(obuf, out_hbm.at[pl.ds(w * _PB, _PB)])


_gather_mean_kernel = functools.partial(
    pl.kernel,
    out_type=jax.ShapeDtypeStruct((_B, _D), jnp.float32),
    mesh=_VMESH,
    scratch_types=[
        pltpu.VMEM((_GT, _CH), jnp.int32),
        pltpu.VMEM((_PB, 16), jnp.float32),
        pltpu.VMEM((6, _CH, _D), jnp.float32),
        pltpu.VMEM((_PB, _D), jnp.float32),
        pltpu.SemaphoreType.DMA,
    ],
)(_gather_mean_body)


@jax.jit
def kernel(x, adj_indices, adj_values, embedding):
    npad = _NNZP - _NNZ
    pad_ids = (jnp.arange(npad, dtype=jnp.int32) * 97) % _N
    sb_shape = (_NCHP // _SB, _SB, _CH)
    rows2 = jnp.concatenate([adj_indices[0], pad_ids]).reshape(sb_shape)
    cols2 = jnp.concatenate([adj_indices[1], pad_ids]).reshape(sb_shape)
    vals2 = jnp.concatenate(
        [adj_values, jnp.zeros((npad,), jnp.float32)]).reshape(sb_shape)

    e0 = jnp.zeros((_M, _D), jnp.float32).at[:_N].set(embedding[1:])
    x_pad = jnp.zeros((_B, _LP), jnp.int32).at[:, :_L].set(x)
    x_pad2 = x_pad.reshape(_B // 2, _CH)

    p1 = _scatter_kernel(e0, rows2, cols2, vals2)
    e1 = _combine_kernel(p1[0], p1[1])
    p2 = _scatter_kernel(e1, rows2, cols2, vals2)
    eavg = _mean_kernel(e0, e1, p2[0], p2[1])
    invb, xg = _prep_kernel(x_pad, x_pad2)
    return _gather_mean_kernel(eavg, xg, invb)
